# Initial kernel scaffold; baseline (speedup 1.0000x reference)
import jax
import jax.numpy as jnp
from jax.experimental import pallas as pl


def kernel(bins, weights, T):
    raise NotImplementedError("write your pallas kernel here")


# speedup vs baseline: 4.0741x; 4.0741x over previous
import functools

import jax
import jax.numpy as jnp
from jax import lax
from jax.experimental import pallas as pl
from jax.experimental.pallas import tpu as pltpu
from jax.experimental.pallas import tpu_sc as plsc

NC = 2
NS = 16
NW = NC * NS
L = 16

T0 = 128
TS = 64
CHUNK = 64


def _body(bins_hbm, w_hbm, u_hbm, out_hbm, bins_v, w_v, cw_v, out_v, u_v):
    n = bins_hbm.shape[0]
    rows_per_w = n // NW
    n_chunks = rows_per_w // CHUNK
    wid = lax.axis_index("s") * NC + lax.axis_index("c")
    base = wid * rows_per_w

    pltpu.sync_copy(u_hbm, u_v)

    def chunk_body(ci, _):
        start = base + ci * CHUNK
        pltpu.sync_copy(bins_hbm.at[pl.ds(start, CHUNK)], bins_v)
        pltpu.sync_copy(w_hbm.at[pl.ds(start, CHUNK)], w_v)

        def row_body(r, _):
            carry = jnp.float32(0.0)
            for i in range(T0 // L):
                w16 = w_v[r, pl.ds(L * i, L)] + jnp.float32(0.01)
                c16 = plsc.cumsum(w16) + carry
                cw_v[r, pl.ds(L * i, L)] = c16
                carry = jnp.max(c16)
            total = carry

            row_idx = jnp.full((L,), r, jnp.int32)
            for b in range(TS // L):
                v = u_v[pl.ds(L * b, L)] * total
                lo = jnp.zeros((L,), jnp.int32)
                for s in (128, 64, 32, 16, 8, 4, 2, 1):
                    cand = lo + s
                    idx = jnp.minimum(cand, T0) - 1
                    val = plsc.load_gather(cw_v, [row_idx, idx])
                    ok = (val <= v) & (cand <= T0)
                    lo = jnp.where(ok, cand, lo)
                c = lo
                bins_g0 = plsc.load_gather(bins_v, [row_idx, c])
                bins_g1 = plsc.load_gather(
                    bins_v, [row_idx, jnp.minimum(c + 1, T0)])
                cg0 = plsc.load_gather(cw_v, [row_idx, jnp.maximum(c - 1, 0)])
                cdf_g0 = jnp.where(c > 0, cg0, jnp.float32(0.0))
                cdf_g1 = plsc.load_gather(
                    cw_v, [row_idx, jnp.minimum(c, T0 - 1)])
                denom = cdf_g1 - cdf_g0
                pos = denom > 0
                t = jnp.where(
                    pos, (v - cdf_g0) / jnp.where(pos, denom, jnp.float32(1.0)),
                    jnp.float32(0.0))
                t = jnp.clip(t, 0.0, 1.0)
                out_v[r, pl.ds(L * b, L)] = bins_g0 + t * (bins_g1 - bins_g0)
            return _

        lax.fori_loop(0, CHUNK, row_body, None)
        pltpu.sync_copy(out_v, out_hbm.at[pl.ds(start, CHUNK)])
        return _

    lax.fori_loop(0, n_chunks, chunk_body, None)


def _sc_sample(bins, weights, u):
    n = bins.shape[0]
    mesh = plsc.VectorSubcoreMesh(
        core_axis_name="c", subcore_axis_name="s", num_cores=NC,
        num_subcores=NS)
    f = pl.kernel(
        _body,
        out_type=jax.ShapeDtypeStruct((n, TS), jnp.float32),
        mesh=mesh,
        scratch_types=[
            pltpu.VMEM((CHUNK, T0 + 1), jnp.float32),
            pltpu.VMEM((CHUNK, T0), jnp.float32),
            pltpu.VMEM((CHUNK, T0), jnp.float32),
            pltpu.VMEM((CHUNK, TS), jnp.float32),
            pltpu.VMEM((TS,), jnp.float32),
        ],
        compiler_params=pltpu.CompilerParams(needs_layout_passes=False),
    )
    return f(bins, weights, u)


def kernel(bins, weights, T):
    tf = jnp.asarray(T, jnp.float32)
    u = 0.5 / tf + jnp.arange(TS, dtype=jnp.float32) * ((1.0 - 1.0 / tf)
                                                        / (TS - 1))
    return _sc_sample(bins, weights, u.astype(jnp.float32))


# speedup vs baseline: 7.2283x; 1.7742x over previous
import jax
import jax.numpy as jnp
from jax import lax
from jax.experimental import pallas as pl
from jax.experimental.pallas import tpu as pltpu
from jax.experimental.pallas import tpu_sc as plsc

NC = 2
NS = 16
NW = NC * NS
L = 16

T0 = 128
TS = 64
CHUNK = 64
RU = 4
HW = 80


def _process_row(r, dr, bins_v, w_v, cw_v, out_v, u_vecs, h_v, ones16):
    row_idx = jnp.full((L,), r, jnp.int32)
    dr_idx = jnp.full((L,), dr, jnp.int32)

    carry = jnp.float32(0.0)
    cw_vecs = []
    for i in range(T0 // L):
        w16 = w_v[r, pl.ds(L * i, L)] + jnp.float32(0.01)
        c16 = plsc.cumsum(w16) + carry
        cw_v[r, pl.ds(L * i, L)] = c16
        cw_vecs.append(c16)
        carry = jnp.max(c16)
    total = carry

    zero16 = jnp.zeros((L,), jnp.int32)
    for i in range(4):
        h_v[dr, pl.ds(L * i, L)] = zero16

    inv = jnp.full((L,), jnp.float32(TS)) / lax.broadcast_in_dim(
        total, (L,), ())
    for i in range(T0 // L):
        t = cw_vecs[i] * inv - jnp.float32(0.5)
        ti = t.astype(jnp.int32)
        m = ti + (ti.astype(jnp.float32) < t).astype(jnp.int32)
        m = jnp.clip(m, 0, TS)
        plsc.addupdate_scatter(h_v, [dr_idx, m], ones16)

    icarry = jnp.int32(0)
    for b in range(TS // L):
        hv = h_v[dr, pl.ds(L * b, L)]
        c = plsc.cumsum(hv) + icarry
        icarry = jnp.max(c)
        v = u_vecs[b] * total
        bins_g0 = plsc.load_gather(bins_v, [row_idx, c])
        bins_g1 = plsc.load_gather(bins_v, [row_idx, jnp.minimum(c + 1, T0)])
        cg0 = plsc.load_gather(cw_v, [row_idx, jnp.maximum(c - 1, 0)])
        cdf_g0 = jnp.where(c > 0, cg0, jnp.float32(0.0))
        cdf_g1 = plsc.load_gather(cw_v, [row_idx, jnp.minimum(c, T0 - 1)])
        denom = cdf_g1 - cdf_g0
        pos = denom > 0
        t = jnp.where(
            pos, (v - cdf_g0) / jnp.where(pos, denom, jnp.float32(1.0)),
            jnp.float32(0.0))
        t = jnp.clip(t, 0.0, 1.0)
        out_v[r, pl.ds(L * b, L)] = bins_g0 + t * (bins_g1 - bins_g0)


def _body(bins_hbm, w_hbm, u_hbm, out_hbm, bins_v, w_v, cw_v, out_v, u_v, h_v):
    n = bins_hbm.shape[0]
    rows_per_w = n // NW
    n_chunks = rows_per_w // CHUNK
    wid = lax.axis_index("s") * NC + lax.axis_index("c")
    base = wid * rows_per_w

    pltpu.sync_copy(u_hbm, u_v)
    ones16 = jnp.ones((L,), jnp.int32)

    def chunk_body(ci, _):
        start = base + ci * CHUNK
        pltpu.sync_copy(bins_hbm.at[pl.ds(start, CHUNK)], bins_v)
        pltpu.sync_copy(w_hbm.at[pl.ds(start, CHUNK)], w_v)
        u_vecs = [u_v[pl.ds(L * b, L)] for b in range(TS // L)]

        def group_body(q, _):
            for dr in range(RU):
                _process_row(q * RU + dr, dr, bins_v, w_v, cw_v, out_v,
                             u_vecs, h_v, ones16)
            return _

        lax.fori_loop(0, CHUNK // RU, group_body, None)
        pltpu.sync_copy(out_v, out_hbm.at[pl.ds(start, CHUNK)])
        return _

    lax.fori_loop(0, n_chunks, chunk_body, None)


def _sc_sample(bins, weights, u):
    n = bins.shape[0]
    mesh = plsc.VectorSubcoreMesh(
        core_axis_name="c", subcore_axis_name="s", num_cores=NC,
        num_subcores=NS)
    f = pl.kernel(
        _body,
        out_type=jax.ShapeDtypeStruct((n, TS), jnp.float32),
        mesh=mesh,
        scratch_types=[
            pltpu.VMEM((CHUNK, T0 + 1), jnp.float32),
            pltpu.VMEM((CHUNK, T0), jnp.float32),
            pltpu.VMEM((CHUNK, T0), jnp.float32),
            pltpu.VMEM((CHUNK, TS), jnp.float32),
            pltpu.VMEM((TS,), jnp.float32),
            pltpu.VMEM((RU, HW), jnp.int32),
        ],
        compiler_params=pltpu.CompilerParams(needs_layout_passes=False),
    )
    return f(bins, weights, u)


def kernel(bins, weights, T):
    tf = jnp.asarray(T, jnp.float32)
    u = 0.5 / tf + jnp.arange(TS, dtype=jnp.float32) * ((1.0 - 1.0 / tf)
                                                        / (TS - 1))
    return _sc_sample(bins, weights, u.astype(jnp.float32))


# speedup vs baseline: 8.0938x; 1.1197x over previous
import jax
import jax.numpy as jnp
from jax import lax
from jax.experimental import pallas as pl
from jax.experimental.pallas import tpu as pltpu
from jax.experimental.pallas import tpu_sc as plsc

NC = 2
NS = 16
NW = NC * NS
L = 16

T0 = 128
TS = 64
CHUNK = 64
RU = 4
HW = 80


def _process_row(r, dr, bins_v, w_v, cw_v, out_v, u_vecs, h_v, ones16):
    row_idx = jnp.full((L,), r, jnp.int32)
    dr_idx = jnp.full((L,), dr, jnp.int32)

    carry = jnp.float32(0.0)
    cw_vecs = []
    for i in range(T0 // L):
        w16 = w_v[r, pl.ds(L * i, L)] + jnp.float32(0.01)
        c16 = plsc.cumsum(w16) + carry
        cw_v[r, pl.ds(L * i, L)] = c16
        cw_vecs.append(c16)
        carry = jnp.max(c16)
    total = carry

    zero16 = jnp.zeros((L,), jnp.int32)
    for i in range(4):
        h_v[dr, pl.ds(L * i, L)] = zero16

    inv = jnp.full((L,), jnp.float32(TS)) / lax.broadcast_in_dim(
        total, (L,), ())
    for i in range(T0 // L):
        t = cw_vecs[i] * inv - jnp.float32(0.5)
        ti = t.astype(jnp.int32)
        m = ti + (ti.astype(jnp.float32) < t).astype(jnp.int32)
        m = jnp.clip(m, 0, TS)
        plsc.addupdate_scatter(h_v, [dr_idx, m], ones16)

    icarry = jnp.int32(0)
    for b in range(TS // L):
        hv = h_v[dr, pl.ds(L * b, L)]
        c = plsc.cumsum(hv) + icarry
        icarry = jnp.max(c)
        v = u_vecs[b] * total
        bins_g0 = plsc.load_gather(bins_v, [row_idx, c])
        bins_g1 = plsc.load_gather(bins_v, [row_idx, jnp.minimum(c + 1, T0)])
        cg0 = plsc.load_gather(cw_v, [row_idx, jnp.maximum(c - 1, 0)])
        cdf_g0 = jnp.where(c > 0, cg0, jnp.float32(0.0))
        cdf_g1 = plsc.load_gather(cw_v, [row_idx, jnp.minimum(c, T0 - 1)])
        denom = cdf_g1 - cdf_g0
        pos = denom > 0
        t = jnp.where(
            pos, (v - cdf_g0) / jnp.where(pos, denom, jnp.float32(1.0)),
            jnp.float32(0.0))
        t = jnp.clip(t, 0.0, 1.0)
        out_v[r, pl.ds(L * b, L)] = bins_g0 + t * (bins_g1 - bins_g0)


def _body(bins_hbm, w_hbm, u_hbm, out_hbm, bins_v, w_v, cw_v, out_v, u_v,
          h_v, sbi, swi, sout):
    n = bins_hbm.shape[0]
    rows_per_w = n // NW
    n_chunks = rows_per_w // CHUNK
    wid = lax.axis_index("s") * NC + lax.axis_index("c")
    base = wid * rows_per_w

    pltpu.sync_copy(u_hbm, u_v)
    ones16 = jnp.ones((L,), jnp.int32)
    u_vecs = [u_v[pl.ds(L * b, L)] for b in range(TS // L)]

    def start_in(ci, buf):
        start = base + ci * CHUNK
        pltpu.async_copy(bins_hbm.at[pl.ds(start, CHUNK)], bins_v.at[buf],
                         sbi[buf])
        pltpu.async_copy(w_hbm.at[pl.ds(start, CHUNK)], w_v.at[buf],
                         swi[buf])

    def wait_in(buf):
        pltpu.make_async_copy(bins_hbm.at[pl.ds(0, CHUNK)], bins_v.at[buf],
                              sbi[buf]).wait()
        pltpu.make_async_copy(w_hbm.at[pl.ds(0, CHUNK)], w_v.at[buf],
                              swi[buf]).wait()

    def wait_out(buf):
        pltpu.make_async_copy(out_v.at[buf], out_hbm.at[pl.ds(0, CHUNK)],
                              sout[buf]).wait()

    def process(ci, buf):
        def group_body(q, _):
            for dr in range(RU):
                _process_row(q * RU + dr, dr, bins_v.at[buf], w_v.at[buf],
                             cw_v, out_v.at[buf], u_vecs, h_v, ones16)
            return _

        lax.fori_loop(0, CHUNK // RU, group_body, None)
        pltpu.async_copy(out_v.at[buf],
                         out_hbm.at[pl.ds(base + ci * CHUNK, CHUNK)],
                         sout[buf])

    start_in(0, 0)

    def chunk_pair(hh, _):
        ci0 = 2 * hh

        @pl.when(hh > 0)
        def _w0():
            wait_out(0)
        start_in(ci0 + 1, 1)
        wait_in(0)
        process(ci0, 0)

        @pl.when(hh > 0)
        def _w1():
            wait_out(1)

        @pl.when(hh < (n_chunks // 2) - 1)
        def _pf():
            start_in(ci0 + 2, 0)
        wait_in(1)
        process(ci0 + 1, 1)
        return _

    lax.fori_loop(0, n_chunks // 2, chunk_pair, None)
    wait_out(0)
    wait_out(1)


def _sc_sample(bins, weights, u):
    n = bins.shape[0]
    mesh = plsc.VectorSubcoreMesh(
        core_axis_name="c", subcore_axis_name="s", num_cores=NC,
        num_subcores=NS)
    f = pl.kernel(
        _body,
        out_type=jax.ShapeDtypeStruct((n, TS), jnp.float32),
        mesh=mesh,
        scratch_types=[
            pltpu.VMEM((2, CHUNK, T0 + 1), jnp.float32),
            pltpu.VMEM((2, CHUNK, T0), jnp.float32),
            pltpu.VMEM((CHUNK, T0), jnp.float32),
            pltpu.VMEM((2, CHUNK, TS), jnp.float32),
            pltpu.VMEM((TS,), jnp.float32),
            pltpu.VMEM((RU, HW), jnp.int32),
            [pltpu.SemaphoreType.DMA] * 2,
            [pltpu.SemaphoreType.DMA] * 2,
            [pltpu.SemaphoreType.DMA] * 2,
        ],
        compiler_params=pltpu.CompilerParams(needs_layout_passes=False),
    )
    return f(bins, weights, u)


def kernel(bins, weights, T):
    tf = jnp.asarray(T, jnp.float32)
    u = 0.5 / tf + jnp.arange(TS, dtype=jnp.float32) * ((1.0 - 1.0 / tf)
                                                        / (TS - 1))
    return _sc_sample(bins, weights, u.astype(jnp.float32))
